# baseline (speedup 1.0000x reference)
import jax
import jax.numpy as jnp
from jax.experimental import pallas as pl

_N = 1_000_000
_C = 16
_B = 2048


def _tf_bits(lo):
    ks0 = jnp.uint32(0)
    ks1 = jnp.uint32(42)
    ks2 = jnp.uint32(0x1BD11BDA ^ 42)
    ks = (ks0, ks1, ks2)
    x0 = jnp.full(lo.shape, ks0, jnp.uint32)
    x1 = lo + ks1
    rots = ((13, 15, 26, 6), (17, 29, 16, 24))
    for i in range(5):
        for r in rots[i % 2]:
            x0 = x0 + x1
            x1 = (x1 << jnp.uint32(r)) | (x1 >> jnp.uint32(32 - r))
            x1 = x0 ^ x1
        x0 = x0 + ks[(i + 1) % 3]
        x1 = x1 + ks[(i + 2) % 3] + jnp.uint32(i + 1)
    return x0 ^ x1


def _body(x_ref, o_ref):
    i = pl.program_id(0)
    lane = jax.lax.broadcasted_iota(jnp.uint32, (_C, _B), 1)
    sub = jax.lax.broadcasted_iota(jnp.uint32, (_C, _B), 0)
    r = jnp.uint32(_B) * jnp.uint32(i) + lane
    lo = r * jnp.uint32(_C) + sub
    bits = _tf_bits(lo)
    xt = x_ref[...].T
    fb = (bits >> jnp.uint32(9)) | jnp.uint32(0x3F800000)
    floats = jax.lax.bitcast_convert_type(fb, jnp.float32) - jnp.float32(1.0)
    u = jnp.maximum(
        jnp.float32(1e-20),
        floats * (jnp.float32(1.0) - jnp.float32(1e-20)) + jnp.float32(1e-20),
    )
    g = -jnp.log(-jnp.log(u) + jnp.float32(1e-20))
    v = xt + g
    best_v = v[0:1, :]
    best_i = jnp.zeros((1, _B), jnp.int32)
    for c in range(1, _C):
        vc = v[c:c + 1, :]
        take = vc > best_v
        best_v = jnp.where(take, vc, best_v)
        best_i = jnp.where(take, jnp.int32(c), best_i)
    o_ref[...] = best_i


def kernel(logits):
    out = pl.pallas_call(
        _body,
        grid=(pl.cdiv(_N, _B),),
        in_specs=[pl.BlockSpec((_B, _C), lambda i: (i, 0))],
        out_specs=pl.BlockSpec((1, _B), lambda i: (0, i)),
        out_shape=jax.ShapeDtypeStruct((1, _N), jnp.int32),
    )(logits)
    return out.reshape(_N)


# speedup vs baseline: 2.4168x; 2.4168x over previous
import jax
import jax.numpy as jnp
from jax.experimental import pallas as pl

_N = 1_000_000
_C = 16
_B = 8192


def _tf_bits(lo):
    ks0 = jnp.uint32(0)
    ks1 = jnp.uint32(42)
    ks2 = jnp.uint32(0x1BD11BDA ^ 42)
    ks = (ks0, ks1, ks2)
    x0 = jnp.full(lo.shape, ks0, jnp.uint32)
    x1 = lo + ks1
    rots = ((13, 15, 26, 6), (17, 29, 16, 24))
    for i in range(5):
        for r in rots[i % 2]:
            x0 = x0 + x1
            x1 = (x1 << jnp.uint32(r)) | (x1 >> jnp.uint32(32 - r))
            x1 = x0 ^ x1
        x0 = x0 + ks[(i + 1) % 3]
        x1 = x1 + ks[(i + 2) % 3] + jnp.uint32(i + 1)
    return x0 ^ x1


def _argmax16(v, sub8):
    a, b = v[0:8, :], v[8:16, :]
    take = b > a
    val = jnp.where(take, b, a)
    idx = jnp.where(take, sub8 + jnp.int32(8), sub8)
    for h in (4, 2, 1):
        va, vb = val[0:h, :], val[h:2 * h, :]
        ia, ib = idx[0:h, :], idx[h:2 * h, :]
        take = vb > va
        val = jnp.where(take, vb, va)
        idx = jnp.where(take, ib, ia)
    return idx


def _body(x_ref, base_ref, o_ref):
    i = pl.program_id(0)
    base = base_ref[...]
    x1 = base + (jnp.uint32(16 * _B) * jnp.uint32(i) + jnp.uint32(42))
    bits = _tf_bits_from_x1(x1)
    fb = (bits >> jnp.uint32(9)) | jnp.uint32(0x3F800000)
    f = jax.lax.bitcast_convert_type(fb, jnp.float32) - jnp.float32(1.0)
    u = f + jnp.float32(1e-20)
    g = -jnp.log(-jnp.log(u))
    v = x_ref[...] + g
    sub8 = (base[0:8, :] & jnp.uint32(15)).astype(jnp.int32)
    o_ref[...] = _argmax16(v, sub8)


def _tf_bits_from_x1(x1):
    ks0 = jnp.uint32(0)
    ks1 = jnp.uint32(42)
    ks2 = jnp.uint32(0x1BD11BDA ^ 42)
    ks = (ks0, ks1, ks2)
    x0 = jnp.zeros_like(x1)
    rots = ((13, 15, 26, 6), (17, 29, 16, 24))
    for i in range(5):
        for r in rots[i % 2]:
            x0 = x0 + x1
            x1 = (x1 << jnp.uint32(r)) | (x1 >> jnp.uint32(32 - r))
            x1 = x0 ^ x1
        x0 = x0 + ks[(i + 1) % 3]
        x1 = x1 + ks[(i + 2) % 3] + jnp.uint32(i + 1)
    return x0 ^ x1


def kernel(logits):
    lt = logits.T
    lane = jax.lax.broadcasted_iota(jnp.uint32, (_C, _B), 1)
    sub = jax.lax.broadcasted_iota(jnp.uint32, (_C, _B), 0)
    base = lane * jnp.uint32(_C) + sub
    out = pl.pallas_call(
        _body,
        grid=(pl.cdiv(_N, _B),),
        in_specs=[
            pl.BlockSpec((_C, _B), lambda i: (0, i)),
            pl.BlockSpec((_C, _B), lambda i: (0, 0)),
        ],
        out_specs=pl.BlockSpec((1, _B), lambda i: (0, i)),
        out_shape=jax.ShapeDtypeStruct((1, _N), jnp.int32),
    )(lt, base)
    return out.reshape(_N)
